# baseline (speedup 1.0000x reference)
import functools

import jax
import jax.numpy as jnp
from jax import lax
from jax.experimental import pallas as pl
from jax.experimental.pallas import tpu as pltpu
from jax.experimental.pallas import tpu_sc as plsc

B = 64
V = 1000000
NC = 2
NS = 16
NW = NC * NS
ROWS_PER_W = B // NW
L = 16

SEG = ROWS_PER_W * V
C = 16000
NCHUNK = SEG // C
H = C // 2
U = 5
NV = H // L // U

NEG_INF = float("-inf")


def _reduce_half(buf, off, a0, a1, a2, a3):
    def rbody(i, ms):
        b0, b1, b2, b3 = ms
        o = off + i * (L * U)
        b0 = jnp.maximum(b0, buf[pl.ds(o + 0 * L, L)])
        b1 = jnp.maximum(b1, buf[pl.ds(o + 1 * L, L)])
        b2 = jnp.maximum(b2, buf[pl.ds(o + 2 * L, L)])
        b3 = jnp.maximum(b3, buf[pl.ds(o + 3 * L, L)])
        b0 = jnp.maximum(b0, buf[pl.ds(o + 4 * L, L)])
        return (b0, b1, b2, b3)

    return lax.fori_loop(0, NV, rbody, (a0, a1, a2, a3))


def _process_chunk(buf, t, ysA, ysB, acc):
    mA, mB, cA, cB = acc
    for hi in range(2):
        h = 2 * t + hi
        isA = h < (V // H)
        isA_b = jnp.full((L,), isA)
        colbase = h * H - jnp.where(isA, 0, V)
        ys = jnp.where(isA, ysA, ysB)

        yl = ys - colbase
        in_half = (yl >= 0) & (yl < H)
        ylc = jnp.minimum(jnp.maximum(yl, 0), H - 1)
        vbase = hi * H + (ylc // L) * L
        lane = ylc - (ylc // L) * L
        v = buf[pl.ds(vbase, L)]
        hit = (lax.iota(jnp.int32, L) == lane) & jnp.full((L,), in_half)
        buf[pl.ds(vbase, L)] = jnp.where(
            hit, jnp.full((L,), NEG_INF, jnp.float32), v)

        a = tuple(jnp.where(isA_b, x, y) for x, y in zip(mA, mB))
        c = jnp.where(isA_b, cA, cB)
        c = jnp.where(hit, v, c)
        a = _reduce_half(buf, hi * H, *a)
        mA = tuple(jnp.where(isA_b, x, y) for x, y in zip(a, mA))
        mB = tuple(jnp.where(isA_b, y, x) for x, y in zip(a, mB))
        cA = jnp.where(isA_b, c, cA)
        cB = jnp.where(isA_b, cB, c)
    return (mA, mB, cA, cB)


def _margin_sc(logits, y):
    mesh = plsc.VectorSubcoreMesh(core_axis_name="c", subcore_axis_name="s")

    @functools.partial(
        pl.kernel,
        mesh=mesh,
        out_type=jax.ShapeDtypeStruct((NW, L), jnp.float32),
        compiler_params=pltpu.CompilerParams(needs_layout_passes=False),
        scratch_types=[
            pltpu.VMEM((C,), jnp.float32),
            pltpu.VMEM((C,), jnp.float32),
            pltpu.VMEM((B,), jnp.int32),
            pltpu.VMEM((L,), jnp.float32),
            pltpu.SemaphoreType.DMA,
            pltpu.SemaphoreType.DMA,
        ],
    )
    def k(logits_hbm, y_hbm, out_hbm, buf0, buf1, ybuf, outbuf, sem0, sem1):
        wid = lax.axis_index("s") * NC + lax.axis_index("c")
        seg = pl.multiple_of(wid * SEG, 128)
        pltpu.sync_copy(y_hbm, ybuf)

        def yscal(r):
            vb = (r // L) * L
            yvec = ybuf[pl.ds(vb, L)]
            return jnp.max(jnp.where(lax.iota(jnp.int32, L) == (r - vb), yvec,
                                     jnp.full((L,), -1, jnp.int32)))

        ysA = yscal(wid * ROWS_PER_W)
        ysB = yscal(wid * ROWS_PER_W + 1)

        def chunk_src(t):
            return logits_hbm.at[pl.ds(pl.multiple_of(seg + t * C, 128), C)]

        pltpu.async_copy(chunk_src(0), buf0, sem0)

        def gbody(g, carry):
            ka = 2 * g
            pltpu.async_copy(chunk_src(ka + 1), buf1, sem1)
            pltpu.make_async_copy(chunk_src(ka), buf0, sem0).wait()
            carry = _process_chunk(buf0, ka, ysA, ysB, carry)
            pltpu.async_copy(chunk_src(ka + 2), buf0, sem0)
            pltpu.make_async_copy(chunk_src(ka + 1), buf1, sem1).wait()
            carry = _process_chunk(buf1, ka + 1, ysA, ysB, carry)
            return carry

        ninf = jnp.full((L,), NEG_INF, jnp.float32)
        init = ((ninf,) * 4, (ninf,) * 4, ninf, ninf)
        acc = lax.fori_loop(0, (NCHUNK - 1) // 2, gbody, init)
        pltpu.make_async_copy(chunk_src(NCHUNK - 1), buf0, sem0).wait()
        mA, mB, cA, cB = _process_chunk(buf0, NCHUNK - 1, ysA, ysB, acc)

        lossA = jnp.max(cA) - jnp.max(
            jnp.maximum(jnp.maximum(mA[0], mA[1]), jnp.maximum(mA[2], mA[3])))
        lossB = jnp.max(cB) - jnp.max(
            jnp.maximum(jnp.maximum(mB[0], mB[1]), jnp.maximum(mB[2], mB[3])))

        io = lax.iota(jnp.int32, L)
        outv = jnp.where(io == 0, jnp.full((L,), lossA, jnp.float32),
                         jnp.where(io == 1, jnp.full((L,), lossB, jnp.float32),
                                   jnp.zeros((L,), jnp.float32)))
        outbuf[...] = outv
        pltpu.sync_copy(outbuf, out_hbm.at[wid])

    return k(logits, y)


def kernel(logits, y):
    out = _margin_sc(logits.reshape(-1), y.astype(jnp.int32))
    return out[:, :ROWS_PER_W].reshape(B)


# speedup vs baseline: 41.0885x; 41.0885x over previous
import functools

import jax
import jax.numpy as jnp
from jax import lax
from jax.experimental import pallas as pl
from jax.experimental.pallas import tpu as pltpu
from jax.experimental.pallas import tpu_sc as plsc

B = 64
V = 1000000
NC = 2
NS = 16
NW = NC * NS
L = 16

RG = 8
NQ = 4
QW = 249984
TAIL = V - NQ * QW
CW = 3968
NCH = QW // CW
NVC = CW // L

NEG_INF = float("-inf")


def _fixup_row(buf, j, row_lo, width, ys, c):
    yl = ys - row_lo
    in_blk = (yl >= 0) & (yl < width)
    ylc = jnp.minimum(jnp.maximum(yl, 0), width - 1)
    vb = (ylc // L) * L
    lane = ylc - vb
    v = buf[j, pl.ds(vb, L)]
    hit = (lax.iota(jnp.int32, L) == lane) & jnp.full((L,), in_blk)
    buf[j, pl.ds(vb, L)] = jnp.where(
        hit, jnp.full((L,), NEG_INF, jnp.float32), v)
    return jnp.where(hit, v, c)


def _process_block(buf, col_lo, ys, accs, cs):
    accs = list(accs)
    cs = list(cs)
    for j in range(RG):
        cs[j] = _fixup_row(buf, j, col_lo, CW, ys[j], cs[j])

    def rbody(i, ms):
        o = i * L
        return tuple(
            jnp.maximum(ms[j], buf[j, pl.ds(o, L)]) for j in range(RG))

    return lax.fori_loop(0, NVC, rbody, tuple(accs)), tuple(cs)


def _margin_sc(logits, tail, y):
    mesh = plsc.VectorSubcoreMesh(core_axis_name="c", subcore_axis_name="s")

    @functools.partial(
        pl.kernel,
        mesh=mesh,
        out_type=jax.ShapeDtypeStruct((NW, L), jnp.float32),
        compiler_params=pltpu.CompilerParams(needs_layout_passes=False),
        scratch_types=[
            pltpu.VMEM((RG, CW), jnp.float32),
            pltpu.VMEM((RG, CW), jnp.float32),
            pltpu.VMEM((RG, TAIL), jnp.float32),
            pltpu.VMEM((B,), jnp.int32),
            pltpu.VMEM((L,), jnp.float32),
            pltpu.SemaphoreType.DMA,
            pltpu.SemaphoreType.DMA,
        ],
    )
    def k(logits_hbm, tail_hbm, y_hbm, out_hbm,
          buf0, buf1, tbuf, ybuf, outbuf, sem0, sem1):
        wid = lax.axis_index("s") * NC + lax.axis_index("c")
        a = wid // NQ
        q = wid - a * NQ
        r0 = pl.multiple_of(a * RG, 8)
        colq = q * QW
        pltpu.sync_copy(y_hbm, ybuf)

        vb = pl.multiple_of((a // 2) * L, 16)
        yvec = ybuf[pl.ds(vb, L)]
        lane0 = (a % 2) * RG
        io = lax.iota(jnp.int32, L)
        ys = [jnp.max(jnp.where(io == (lane0 + j), yvec,
                                jnp.full((L,), -1, jnp.int32)))
              for j in range(RG)]

        def blk_src(t):
            return logits_hbm.at[pl.ds(r0, RG),
                                 pl.ds(pl.multiple_of(colq + t * CW, 128), CW)]

        pltpu.async_copy(blk_src(0), buf0, sem0)

        def gbody(g, carry):
            accs, cs = carry
            ka = 2 * g
            pltpu.async_copy(blk_src(ka + 1), buf1, sem1)
            pltpu.make_async_copy(blk_src(ka), buf0, sem0).wait()
            accs, cs = _process_block(buf0, colq + ka * CW, ys, accs, cs)
            pltpu.async_copy(blk_src(ka + 2), buf0, sem0)
            pltpu.make_async_copy(blk_src(ka + 1), buf1, sem1).wait()
            accs, cs = _process_block(buf1, colq + (ka + 1) * CW, ys, accs, cs)
            return (accs, cs)

        ninf = jnp.full((L,), NEG_INF, jnp.float32)
        init = ((ninf,) * RG, (ninf,) * RG)
        accs, cs = lax.fori_loop(0, (NCH - 1) // 2, gbody, init)
        pltpu.make_async_copy(blk_src(NCH - 1), buf0, sem0).wait()
        accs, cs = _process_block(buf0, colq + (NCH - 1) * CW, ys, accs, cs)
        accs = list(accs)
        cs = list(cs)

        pltpu.sync_copy(tail_hbm.at[pl.ds(r0, RG), :], tbuf)
        for j in range(RG):
            cs[j] = _fixup_row(tbuf, j, NQ * QW, TAIL, ys[j], cs[j])
            for u in range(TAIL // L):
                accs[j] = jnp.maximum(accs[j], tbuf[j, pl.ds(u * L, L)])

        outv = jnp.zeros((L,), jnp.float32)
        for j in range(RG):
            outv = jnp.where(io == j,
                             jnp.full((L,), jnp.max(accs[j]), jnp.float32),
                             outv)
            outv = jnp.where(io == (RG + j),
                             jnp.full((L,), jnp.max(cs[j]), jnp.float32),
                             outv)
        outbuf[...] = outv
        pltpu.sync_copy(outbuf, out_hbm.at[wid])

    return k(logits, tail, y)


def kernel(logits, y):
    tail = lax.slice(logits, (0, NQ * QW), (B, V))
    out = _margin_sc(logits, tail, y.astype(jnp.int32))
    p = out.reshape(B // RG, NQ, L)
    m = jnp.max(p[:, :, :RG], axis=1)
    c = jnp.max(p[:, :, RG:], axis=1)
    return (c - m).reshape(B)


# speedup vs baseline: 46.9102x; 1.1417x over previous
import functools

import jax
import jax.numpy as jnp
from jax import lax
from jax.experimental import pallas as pl
from jax.experimental.pallas import tpu as pltpu
from jax.experimental.pallas import tpu_sc as plsc

B = 64
V = 1000000
NC = 2
NS = 16
NW = NC * NS
L = 16

RG = 8
NQ = 4
QW = 249984
TAIL = V - NQ * QW
CW = 3968
NCH = QW // CW
NVC = CW // L

NEG_INF = float("-inf")


def _fixup_row(buf, j, row_lo, width, ys, c):
    yl = ys - row_lo
    in_blk = (yl >= 0) & (yl < width)
    ylc = jnp.minimum(jnp.maximum(yl, 0), width - 1)
    vb = (ylc // L) * L
    lane = ylc - vb
    v = buf[j, pl.ds(vb, L)]
    hit = (lax.iota(jnp.int32, L) == lane) & jnp.full((L,), in_blk)
    buf[j, pl.ds(vb, L)] = jnp.where(
        hit, jnp.full((L,), NEG_INF, jnp.float32), v)
    return jnp.where(hit, v, c)


def _process_block(buf, col_lo, ys, accs, cs):
    accs = list(accs)
    cs = list(cs)
    for j in range(RG):
        cs[j] = _fixup_row(buf, j, col_lo, CW, ys[j], cs[j])

    def rbody(i, ms):
        o = i * L
        return tuple(
            jnp.maximum(ms[j], buf[j, pl.ds(o, L)]) for j in range(RG))

    return lax.fori_loop(0, NVC, rbody, tuple(accs)), tuple(cs)


def _margin_sc(logits, tail, y):
    mesh = plsc.VectorSubcoreMesh(core_axis_name="c", subcore_axis_name="s")

    @functools.partial(
        pl.kernel,
        mesh=mesh,
        out_type=jax.ShapeDtypeStruct((NW, L), jnp.float32),
        compiler_params=pltpu.CompilerParams(needs_layout_passes=False),
        scratch_types=[
            pltpu.VMEM((RG, CW), jnp.float32),
            pltpu.VMEM((RG, CW), jnp.float32),
            pltpu.VMEM((RG, CW), jnp.float32),
            pltpu.VMEM((RG, TAIL), jnp.float32),
            pltpu.VMEM((B,), jnp.int32),
            pltpu.VMEM((L,), jnp.float32),
            pltpu.SemaphoreType.DMA,
            pltpu.SemaphoreType.DMA,
            pltpu.SemaphoreType.DMA,
        ],
    )
    def k(logits_hbm, tail_hbm, y_hbm, out_hbm,
          buf0, buf1, buf2, tbuf, ybuf, outbuf, sem0, sem1, sem2):
        wid = lax.axis_index("s") * NC + lax.axis_index("c")
        a = wid // NQ
        q = wid - a * NQ
        r0 = pl.multiple_of(a * RG, 8)
        colq = q * QW
        pltpu.sync_copy(y_hbm, ybuf)

        vb = pl.multiple_of((a // 2) * L, 16)
        yvec = ybuf[pl.ds(vb, L)]
        lane0 = (a % 2) * RG
        io = lax.iota(jnp.int32, L)
        ys = [jnp.max(jnp.where(io == (lane0 + j), yvec,
                                jnp.full((L,), -1, jnp.int32)))
              for j in range(RG)]

        def blk_src(t):
            return logits_hbm.at[pl.ds(r0, RG),
                                 pl.ds(pl.multiple_of(colq + t * CW, 128), CW)]

        bufs = (buf0, buf1, buf2)
        sems = (sem0, sem1, sem2)
        for t in range(3):
            pltpu.async_copy(blk_src(t), bufs[t], sems[t])

        def gbody(g, carry):
            accs, cs = carry
            ka = 3 * g
            for bi in range(3):
                t = ka + bi
                pltpu.make_async_copy(blk_src(t), bufs[bi], sems[bi]).wait()
                accs, cs = _process_block(
                    bufs[bi], colq + t * CW, ys, accs, cs)
                pltpu.async_copy(blk_src(t + 3), bufs[bi], sems[bi])
            return (accs, cs)

        ninf = jnp.full((L,), NEG_INF, jnp.float32)
        init = ((ninf,) * RG, (ninf,) * RG)
        accs, cs = lax.fori_loop(0, NCH // 3 - 1, gbody, init)
        for bi in range(3):
            t = NCH - 3 + bi
            pltpu.make_async_copy(blk_src(t), bufs[bi], sems[bi]).wait()
            accs, cs = _process_block(bufs[bi], colq + t * CW, ys, accs, cs)
        accs = list(accs)
        cs = list(cs)

        pltpu.sync_copy(tail_hbm.at[pl.ds(r0, RG), :], tbuf)
        for j in range(RG):
            cs[j] = _fixup_row(tbuf, j, NQ * QW, TAIL, ys[j], cs[j])
            for u in range(TAIL // L):
                accs[j] = jnp.maximum(accs[j], tbuf[j, pl.ds(u * L, L)])

        outv = jnp.zeros((L,), jnp.float32)
        for j in range(RG):
            outv = jnp.where(io == j,
                             jnp.full((L,), jnp.max(accs[j]), jnp.float32),
                             outv)
            outv = jnp.where(io == (RG + j),
                             jnp.full((L,), jnp.max(cs[j]), jnp.float32),
                             outv)
        outbuf[...] = outv
        pltpu.sync_copy(outbuf, out_hbm.at[wid])

    return k(logits, tail, y)


def kernel(logits, y):
    tail = lax.slice(logits, (0, NQ * QW), (B, V))
    out = _margin_sc(logits, tail, y.astype(jnp.int32))
    p = out.reshape(B // RG, NQ, L)
    m = jnp.max(p[:, :, :RG], axis=1)
    c = jnp.max(p[:, :, RG:], axis=1)
    return (c - m).reshape(B)
